# baseline (speedup 1.0000x reference)
import functools

import jax
import jax.numpy as jnp
from jax import lax
from jax.experimental import pallas as pl
from jax.experimental.pallas import tpu as pltpu
from jax.experimental.pallas import tpu_sc as plsc

_NC = 2
_NS = 16
_NW = _NC * _NS
_L = 16

_RBLOG = 11
_RB = 1 << _RBLOG
_CW = 128
_IC = 8


def kernel(X, table):
    B0, S = X.shape
    V, D = table.shape
    B = B0 * S
    nblk = (V + _RB - 1) // _RB
    idx2d = X.reshape(B // 128, 128)
    irows = idx2d.shape[0] // _NW
    rows_per_w = B // _NW
    ichunks = irows // _IC
    cap_rows = (rows_per_w + nblk * (_CW - 1)) // _CW + 1
    per_tile_rb = _RB // _NS

    mesh = plsc.VectorSubcoreMesh(core_axis_name="c", subcore_axis_name="s")

    @functools.partial(
        pl.kernel,
        mesh=mesh,
        compiler_params=pltpu.CompilerParams(needs_layout_passes=False),
        out_type=jax.ShapeDtypeStruct((B, D), jnp.float32),
        scratch_types=[
            pltpu.VMEM((2, _IC, 128), jnp.int32),
            pltpu.VMEM((cap_rows, _CW), jnp.int32),
            pltpu.VMEM((nblk, _L), jnp.int32),
            pltpu.VMEM((nblk * _L,), jnp.int32),
            pltpu.VMEM((2, _CW, D), jnp.float32),
            pltpu.VMEM((2, _CW), jnp.int32),
            pltpu.VMEM((2, _CW), jnp.int32),
            pltpu.VMEM_SHARED((_RB, D), jnp.float32),
            pltpu.VMEM_SHARED((_RB, D), jnp.float32),
            pltpu.SMEM((128,), jnp.int32),
            pltpu.SemaphoreType.DMA,
            pltpu.SemaphoreType.DMA,
            pltpu.SemaphoreType.DMA,
            pltpu.SemaphoreType.DMA,
        ],
    )
    def gather_kernel(idx_hbm, table_hbm, out_hbm, idxc, bins, hist2d,
                      laneoff, rowbufs, lidxb, posb, blk0, blk1, smem,
                      ssem, gsem, osem, isem):
        cid = lax.axis_index("c")
        sid = lax.axis_index("s")
        wid = sid * _NC + cid
        ibase = wid * irows
        obase = wid * rows_per_w
        iota = lax.iota(jnp.int32, _L)
        ones = jnp.ones((_L,), jnp.int32)
        blks = (blk0, blk1)

        def fire_idx(k):
            pltpu.async_copy(idx_hbm.at[pl.ds(ibase + k * _IC, _IC)],
                             idxc.at[k & 1], isem)

        def drain_idx():
            pltpu.make_async_copy(idx_hbm.at[pl.ds(0, _IC)], idxc.at[0],
                                  isem).wait()

        def stream_pass(per_vreg):
            fire_idx(0)

            def body(k, carry):
                drain_idx()

                @pl.when(k < ichunks - 1)
                def _():
                    fire_idx(k + 1)

                kb = k & 1
                for rr in range(_IC):
                    for c in range(128 // _L):
                        vec = idxc[kb, rr, pl.ds(c * _L, _L)]
                        per_vreg(k * _IC + rr, c, vec)
                return carry

            lax.fori_loop(0, ichunks, body, 0)

        for b in range(nblk):
            hist2d[b, :] = jnp.zeros((_L,), jnp.int32)

        def p1(r, c, vec):
            bid = lax.shift_right_logical(vec, _RBLOG)
            plsc.addupdate_scatter(hist2d, [bid, iota], ones)

        stream_pass(p1)

        start = jnp.int32(0)
        for b in range(nblk):
            h = hist2d[b, :]
            csum = plsc.cumsum(h)
            cnt = csum[_L - 1]
            laneoff[pl.ds(b * _L, _L)] = start + (csum - h)
            nch = (cnt + _CW - 1) >> 7
            smem[2 * b] = start >> 7
            smem[2 * b + 1] = nch
            start = start + (nch << 7)

        def p2(r, c, vec):
            bid = lax.shift_right_logical(vec, _RBLOG)
            key = (bid << 4) | iota
            dest = plsc.load_gather(laneoff, [key])
            pos = obase + r * 128 + c * _L + iota
            packed = ((vec & (_RB - 1)) << 20) | pos
            plsc.store_scatter(bins, [dest >> 7, dest & (_CW - 1)], packed)
            plsc.addupdate_scatter(laneoff, [key], ones)

        stream_pass(p2)

        for b in range(nblk):
            srow = smem[2 * b]
            nch = smem[2 * b + 1]
            lo = laneoff[pl.ds(b * _L, _L)]
            endpos = lo[_L - 1]
            regend = (srow + nch) << 7
            first = bins[srow, pl.ds(0, _L)]
            pv = jnp.broadcast_to(first[0], (_L,))
            for t in range(_CW // _L):
                addr = endpos + t * _L + iota
                m = addr < regend
                plsc.store_scatter(bins, [addr >> 7, addr & (_CW - 1)],
                                   pv, mask=m)

        last_rows = V - (nblk - 1) * _RB
        lt = last_rows // 4

        def fire_stage(b, buf):
            if b == nblk - 1:
                @pl.when(sid < 4)
                def _():
                    pltpu.sync_copy(
                        table_hbm.at[pl.ds(b * _RB + sid * lt, lt)],
                        buf.at[pl.ds(sid * lt, lt)])
                return None
            return pltpu.async_copy(
                table_hbm.at[pl.ds(b * _RB + sid * per_tile_rb, per_tile_rb)],
                buf.at[pl.ds(sid * per_tile_rb, per_tile_rb)], ssem)

        def drain_scatter():
            pltpu.make_async_copy(
                out_hbm.at[pl.ds(0, _CW)], rowbufs.at[0], osem).wait()

        stage_h = fire_stage(0, blks[0])
        for b in range(nblk):
            cur = blks[b % 2]
            if stage_h is not None:
                stage_h.wait()
            plsc.subcore_barrier()
            if b + 1 < nblk:
                stage_h = fire_stage(b + 1, blks[(b + 1) % 2])

            srow = smem[2 * b]
            nch = smem[2 * b + 1]

            def chunk_body(j, carry, cur=cur, srow=srow):
                jb = j & 1

                @pl.when(j >= 2)
                def _():
                    drain_scatter()

                for c in range(128 // _L):
                    pk = bins[srow + j, pl.ds(c * _L, _L)]
                    lidxb[jb, pl.ds(c * _L, _L)] = lax.shift_right_logical(
                        pk, 20)
                    posb[jb, pl.ds(c * _L, _L)] = pk & 0xFFFFF
                pltpu.async_copy(
                    cur.at[lidxb.at[jb]], rowbufs.at[jb], gsem).wait()
                pltpu.async_copy(
                    rowbufs.at[jb], out_hbm.at[posb.at[jb]], osem)
                return carry

            lax.fori_loop(0, nch, chunk_body, 0)

            def tail_drain(j, carry):
                drain_scatter()
                return carry

            lax.fori_loop(0, jnp.minimum(nch, 2), tail_drain, 0)

    out = gather_kernel(idx2d, table)
    return out.reshape(B0, S, D)


# speedup vs baseline: 1.0219x; 1.0219x over previous
import functools

import jax
import jax.numpy as jnp
from jax import lax
from jax.experimental import pallas as pl
from jax.experimental.pallas import tpu as pltpu
from jax.experimental.pallas import tpu_sc as plsc

_NC = 2
_NS = 16
_NW = _NC * _NS
_L = 16

_RBLOG = 11
_RB = 1 << _RBLOG
_CW = 128
_IC = 8


def kernel(X, table):
    B0, S = X.shape
    V, D = table.shape
    B = B0 * S
    nblk = (V + _RB - 1) // _RB
    idx2d = X.reshape(B // 128, 128)
    irows = idx2d.shape[0] // _NW
    rows_per_w = B // _NW
    ichunks = irows // _IC
    cap_rows = (rows_per_w + nblk * (_CW - 1)) // _CW + 1
    per_tile_rb = _RB // _NS

    mesh = plsc.VectorSubcoreMesh(core_axis_name="c", subcore_axis_name="s")

    @functools.partial(
        pl.kernel,
        mesh=mesh,
        compiler_params=pltpu.CompilerParams(needs_layout_passes=False),
        out_type=jax.ShapeDtypeStruct((B, D), jnp.float32),
        scratch_types=[
            pltpu.VMEM((2, _IC, 128), jnp.int32),
            pltpu.VMEM((cap_rows, _CW), jnp.int32),
            pltpu.VMEM((nblk, _L), jnp.int32),
            pltpu.VMEM((nblk * _L,), jnp.int32),
            pltpu.VMEM((2, _CW, D), jnp.float32),
            pltpu.VMEM((2, _CW), jnp.int32),
            pltpu.VMEM((2, _CW), jnp.int32),
            pltpu.VMEM_SHARED((_RB, D), jnp.float32),
            pltpu.VMEM_SHARED((_RB, D), jnp.float32),
            pltpu.SMEM((128,), jnp.int32),
            pltpu.SemaphoreType.DMA,
            pltpu.SemaphoreType.DMA,
            pltpu.SemaphoreType.DMA,
            pltpu.SemaphoreType.DMA,
        ],
    )
    def gather_kernel(idx_hbm, table_hbm, out_hbm, idxc, bins, hist2d,
                      laneoff, rowbufs, lidxb, posb, blk0, blk1, smem,
                      ssem, gsem, osem, isem):
        cid = lax.axis_index("c")
        sid = lax.axis_index("s")
        wid = sid * _NC + cid
        ibase = wid * irows
        obase = wid * rows_per_w
        iota = lax.iota(jnp.int32, _L)
        ones = jnp.ones((_L,), jnp.int32)
        blks = (blk0, blk1)

        def fire_idx(k):
            pltpu.async_copy(idx_hbm.at[pl.ds(ibase + k * _IC, _IC)],
                             idxc.at[k & 1], isem)

        def drain_idx():
            pltpu.make_async_copy(idx_hbm.at[pl.ds(0, _IC)], idxc.at[0],
                                  isem).wait()

        def stream_pass(per_vreg):
            fire_idx(0)

            def body(k, carry):
                drain_idx()

                @pl.when(k < ichunks - 1)
                def _():
                    fire_idx(k + 1)

                kb = k & 1
                for rr in range(_IC):
                    for c in range(128 // _L):
                        vec = idxc[kb, rr, pl.ds(c * _L, _L)]
                        per_vreg(k * _IC + rr, c, vec)
                return carry

            lax.fori_loop(0, ichunks, body, 0)

        for b in range(nblk):
            hist2d[b, :] = jnp.zeros((_L,), jnp.int32)

        def p1(r, c, vec):
            bid = lax.shift_right_logical(vec, _RBLOG)
            plsc.addupdate_scatter(hist2d, [bid, iota], ones)

        stream_pass(p1)

        start = jnp.int32(0)
        for b in range(nblk):
            h = hist2d[b, :]
            csum = plsc.cumsum(h)
            cnt = csum[_L - 1]
            laneoff[pl.ds(b * _L, _L)] = start + (csum - h)
            nch = (cnt + _CW - 1) >> 7
            smem[2 * b] = start >> 7
            smem[2 * b + 1] = nch
            start = start + (nch << 7)

        def p2(r, c, vec):
            bid = lax.shift_right_logical(vec, _RBLOG)
            key = (bid << 4) | iota
            dest = plsc.load_gather(laneoff, [key])
            pos = obase + r * 128 + c * _L + iota
            packed = ((vec & (_RB - 1)) << 20) | pos
            plsc.store_scatter(bins, [dest >> 7, dest & (_CW - 1)], packed)
            plsc.addupdate_scatter(laneoff, [key], ones)

        stream_pass(p2)

        for b in range(nblk):
            srow = smem[2 * b]
            nch = smem[2 * b + 1]
            lo = laneoff[pl.ds(b * _L, _L)]
            endpos = lo[_L - 1]
            regend = (srow + nch) << 7
            first = bins[srow, pl.ds(0, _L)]
            pv = jnp.broadcast_to(first[0], (_L,))
            for t in range(_CW // _L):
                addr = endpos + t * _L + iota
                m = addr < regend
                plsc.store_scatter(bins, [addr >> 7, addr & (_CW - 1)],
                                   pv, mask=m)

        last_rows = V - (nblk - 1) * _RB
        lt = last_rows // 4

        def fire_stage(b, buf):
            pltpu.async_copy(
                table_hbm.at[pl.ds(b * _RB + sid * per_tile_rb, per_tile_rb)],
                buf.at[pl.ds(sid * per_tile_rb, per_tile_rb)], ssem)

        def stage_last(buf):
            @pl.when(sid < 4)
            def _():
                pltpu.sync_copy(
                    table_hbm.at[pl.ds((nblk - 1) * _RB + sid * lt, lt)],
                    buf.at[pl.ds(sid * lt, lt)])

        def drain_stage():
            pltpu.make_async_copy(
                table_hbm.at[pl.ds(0, per_tile_rb)],
                blk0.at[pl.ds(0, per_tile_rb)], ssem).wait()

        def drain_scatter():
            pltpu.make_async_copy(
                out_hbm.at[pl.ds(0, _CW)], rowbufs.at[0], osem).wait()

        def drain_gather():
            pltpu.make_async_copy(
                out_hbm.at[pl.ds(0, _CW)], rowbufs.at[0], gsem).wait()

        def process_block(b, cur):
            srow = smem[2 * b]
            nch = smem[2 * b + 1]

            def body(j, carry):
                jb = j & 1

                @pl.when(j < nch)
                def _():
                    @pl.when(j >= 2)
                    def _():
                        drain_scatter()

                    for c in range(128 // _L):
                        pk = bins[srow + j, pl.ds(c * _L, _L)]
                        lidxb[jb, pl.ds(c * _L, _L)] = (
                            lax.shift_right_logical(pk, 20))
                        posb[jb, pl.ds(c * _L, _L)] = pk & 0xFFFFF
                    pltpu.async_copy(cur.at[lidxb.at[jb]], rowbufs.at[jb],
                                     gsem)

                @pl.when(j >= 1)
                def _():
                    pj = (j - 1) & 1
                    drain_gather()
                    pltpu.async_copy(
                        rowbufs.at[pj], out_hbm.at[posb.at[pj]], osem)
                return carry

            lax.fori_loop(0, nch + 1, body, 0)

            def tail_drain(j, carry):
                drain_scatter()
                return carry

            lax.fori_loop(0, jnp.minimum(nch, 2), tail_drain, 0)

        def block_step(b, cur, nxt, last_next):
            drain_stage()
            plsc.subcore_barrier()

            @pl.when(b < nblk - 2)
            def _():
                fire_stage(b + 1, nxt)

            @pl.when(last_next)
            def _():
                stage_last(nxt)

            process_block(b, cur)

        fire_stage(0, blks[0])

        def pair_body(i, carry):
            b0 = 2 * i
            block_step(b0, blks[0], blks[1], jnp.bool_(False))
            block_step(b0 + 1, blks[1], blks[0], b0 + 1 == nblk - 2)
            return carry

        lax.fori_loop(0, (nblk - 1) // 2, pair_body, 0)

        plsc.subcore_barrier()
        process_block(nblk - 1, blks[0])

    out = gather_kernel(idx2d, table)
    return out.reshape(B0, S, D)


# speedup vs baseline: 1.5179x; 1.4854x over previous
import functools

import jax
import jax.numpy as jnp
from jax import lax
from jax.experimental import pallas as pl
from jax.experimental.pallas import tpu as pltpu
from jax.experimental.pallas import tpu_sc as plsc

_NC = 2
_NS = 16
_NW = _NC * _NS

_CH = 128
_NBUF = 5


def kernel(X, table):
    B0, S = X.shape
    V, D = table.shape
    B = B0 * S
    idx2d = X.reshape(B // _CH, _CH)
    irows_per_w = idx2d.shape[0] // _NW
    rows_per_w = B // _NW
    chunks = rows_per_w // _CH
    rounds = chunks // _NBUF

    mesh = plsc.VectorSubcoreMesh(core_axis_name="c", subcore_axis_name="s")

    @functools.partial(
        pl.kernel,
        mesh=mesh,
        out_type=jax.ShapeDtypeStruct((B, D), jnp.float32),
        scratch_types=[
            pltpu.VMEM((irows_per_w, _CH), jnp.int32),
            pltpu.VMEM((_NBUF, _CH, D), jnp.float32),
        ] + [pltpu.SemaphoreType.DMA] * (2 * _NBUF),
    )
    def gather_kernel(idx_hbm, table_hbm, out_hbm, idx_v, rows_v, *sems):
        gsems = sems[:_NBUF]
        osems = sems[_NBUF:]
        wid = lax.axis_index("s") * _NC + lax.axis_index("c")
        ibase = wid * irows_per_w
        obase = wid * rows_per_w
        pltpu.sync_copy(idx_hbm.at[pl.ds(ibase, irows_per_w)], idx_v)

        def fire_gather(i, b):
            pltpu.async_copy(table_hbm.at[idx_v.at[i]], rows_v.at[b], gsems[b])

        def wait_gather(b):
            pltpu.make_async_copy(
                out_hbm.at[pl.ds(obase, _CH)], rows_v.at[b], gsems[b],
            ).wait()

        def fire_out(i, b):
            return pltpu.async_copy(
                rows_v.at[b], out_hbm.at[pl.ds(obase + i * _CH, _CH)], osems[b])

        for b in range(_NBUF):
            fire_gather(b, b)

        def body(j, carry):
            i0 = _NBUF * j
            hs = []
            for b in range(_NBUF):
                wait_gather(b)
                hs.append(fire_out(i0 + b, b))
            for b in range(_NBUF):
                hs[b].wait()
                fire_gather(i0 + _NBUF + b, b)
            return carry

        lax.fori_loop(0, rounds - 1, body, 0)

        hs = []
        for b in range(_NBUF):
            wait_gather(b)
            hs.append(fire_out(chunks - _NBUF + b, b))
        for b in range(_NBUF):
            hs[b].wait()

    out = gather_kernel(idx2d, table)
    return out.reshape(B0, S, D)
